# baseline (speedup 1.0000x reference)
import jax
import jax.numpy as jnp
from jax.experimental import pallas as pl
from jax.experimental.pallas import tpu as pltpu

_R = 1024
_NBUF = 8
_LOOK = 4


def _make_body(batch, seq, dim):
    s_ch = seq // _R
    steps = [(s, b) for s in range(s_ch) for b in range(batch)]
    T = len(steps)

    def body(x_hbm, pos_hbm, o_hbm, xb, pb, insem, psem, outsem):
        def fire_xin(t):
            s, b = steps[t]
            pltpu.make_async_copy(
                x_hbm.at[b, pl.ds(s * _R, _R), :], xb.at[t % _NBUF],
                insem.at[t % _NBUF],
            ).start()

        def wait_xin(t):
            pltpu.make_async_copy(
                x_hbm.at[0, pl.ds(0, _R), :], xb.at[t % _NBUF],
                insem.at[t % _NBUF],
            ).wait()

        def fire_pin(s):
            pltpu.make_async_copy(
                pos_hbm.at[pl.ds(s * _R, _R), :], pb.at[s % 2], psem.at[s % 2]
            ).start()

        def wait_pin(s):
            pltpu.make_async_copy(
                pos_hbm.at[pl.ds(0, _R), :], pb.at[s % 2], psem.at[s % 2]
            ).wait()

        def fire_out(t):
            s, b = steps[t]
            pltpu.make_async_copy(
                xb.at[t % _NBUF], o_hbm.at[b, pl.ds(s * _R, _R), :],
                outsem.at[t % _NBUF],
            ).start()

        def wait_out(t):
            pltpu.make_async_copy(
                xb.at[t % _NBUF], o_hbm.at[0, pl.ds(0, _R), :],
                outsem.at[t % _NBUF],
            ).wait()

        fire_pin(0)
        fire_pin(1)
        for t in range(_LOOK):
            fire_xin(t)

        last_waited = -1
        for t in range(T):
            if t + _LOOK < T:
                if t - _NBUF + _LOOK >= 0:
                    wait_out(t - _NBUF + _LOOK)
                    last_waited = t - _NBUF + _LOOK
                fire_xin(t + _LOOK)
            s, b = steps[t]
            if b == 0:
                wait_pin(s)
            wait_xin(t)
            xb[t % _NBUF] = xb[t % _NBUF] + pb[s % 2]
            fire_out(t)
            if b == batch - 1 and s + 2 < s_ch:
                fire_pin(s + 2)

        for t in range(last_waited + 1, T):
            wait_out(t)

    return body


def kernel(x, pos_table):
    batch, seq, dim = x.shape
    body = _make_body(batch, seq, dim)
    return pl.pallas_call(
        body,
        in_specs=[
            pl.BlockSpec(memory_space=pl.ANY),
            pl.BlockSpec(memory_space=pl.ANY),
        ],
        out_specs=pl.BlockSpec(memory_space=pl.ANY),
        out_shape=jax.ShapeDtypeStruct((batch, seq, dim), x.dtype),
        scratch_shapes=[
            pltpu.VMEM((_NBUF, _R, dim), jnp.float32),
            pltpu.VMEM((2, _R, dim), jnp.float32),
            pltpu.SemaphoreType.DMA((_NBUF,)),
            pltpu.SemaphoreType.DMA((2,)),
            pltpu.SemaphoreType.DMA((_NBUF,)),
        ],
    )(x, pos_table)


# speedup vs baseline: 1.0020x; 1.0020x over previous
import jax
import jax.numpy as jnp
from jax.experimental import pallas as pl
from jax.experimental.pallas import tpu as pltpu

_R = 2048
_NBUF = 4
_LOOK = 3


def _make_body(batch, seq, dim):
    s_ch = seq // _R
    steps = [(s, b) for s in range(s_ch) for b in range(batch)]
    T = len(steps)

    def body(x_hbm, pos_hbm, o_hbm, xb, pb, insem, psem, outsem):
        def fire_xin(t):
            s, b = steps[t]
            pltpu.make_async_copy(
                x_hbm.at[b, pl.ds(s * _R, _R), :], xb.at[t % _NBUF],
                insem.at[t % _NBUF],
            ).start()

        def wait_xin(t):
            pltpu.make_async_copy(
                x_hbm.at[0, pl.ds(0, _R), :], xb.at[t % _NBUF],
                insem.at[t % _NBUF],
            ).wait()

        def fire_pin(s):
            pltpu.make_async_copy(
                pos_hbm.at[pl.ds(s * _R, _R), :], pb.at[s % 2], psem.at[s % 2]
            ).start()

        def wait_pin(s):
            pltpu.make_async_copy(
                pos_hbm.at[pl.ds(0, _R), :], pb.at[s % 2], psem.at[s % 2]
            ).wait()

        def fire_out(t):
            s, b = steps[t]
            pltpu.make_async_copy(
                xb.at[t % _NBUF], o_hbm.at[b, pl.ds(s * _R, _R), :],
                outsem.at[t % _NBUF],
            ).start()

        def wait_out(t):
            pltpu.make_async_copy(
                xb.at[t % _NBUF], o_hbm.at[0, pl.ds(0, _R), :],
                outsem.at[t % _NBUF],
            ).wait()

        fire_pin(0)
        fire_pin(1)
        for t in range(_LOOK):
            fire_xin(t)

        last_waited = -1
        for t in range(T):
            if t + _LOOK < T:
                if t - _NBUF + _LOOK >= 0:
                    wait_out(t - _NBUF + _LOOK)
                    last_waited = t - _NBUF + _LOOK
                fire_xin(t + _LOOK)
            s, b = steps[t]
            if b == 0:
                wait_pin(s)
            wait_xin(t)
            xb[t % _NBUF] = xb[t % _NBUF] + pb[s % 2]
            fire_out(t)
            if b == batch - 1 and s + 2 < s_ch:
                fire_pin(s + 2)

        for t in range(last_waited + 1, T):
            wait_out(t)

    return body


def kernel(x, pos_table):
    batch, seq, dim = x.shape
    body = _make_body(batch, seq, dim)
    return pl.pallas_call(
        body,
        in_specs=[
            pl.BlockSpec(memory_space=pl.ANY),
            pl.BlockSpec(memory_space=pl.ANY),
        ],
        out_specs=pl.BlockSpec(memory_space=pl.ANY),
        out_shape=jax.ShapeDtypeStruct((batch, seq, dim), x.dtype),
        scratch_shapes=[
            pltpu.VMEM((_NBUF, _R, dim), jnp.float32),
            pltpu.VMEM((2, _R, dim), jnp.float32),
            pltpu.SemaphoreType.DMA((_NBUF,)),
            pltpu.SemaphoreType.DMA((2,)),
            pltpu.SemaphoreType.DMA((_NBUF,)),
        ],
    )(x, pos_table)


# speedup vs baseline: 1.0056x; 1.0035x over previous
import jax
import jax.numpy as jnp
from jax.experimental import pallas as pl
from jax.experimental.pallas import tpu as pltpu

_R = 2048
_NBUF = 4
_LOOK = 2


def _make_body(batch, seq, dim):
    s_ch = seq // _R
    steps = [(s, b) for s in range(s_ch) for b in range(batch)]
    T = len(steps)

    def body(x_hbm, pos_hbm, o_hbm, xb, pb, insem, psem, outsem):
        def fire_xin(t):
            s, b = steps[t]
            pltpu.make_async_copy(
                x_hbm.at[b, pl.ds(s * _R, _R), :], xb.at[t % _NBUF],
                insem.at[t % _NBUF],
            ).start()

        def wait_xin(t):
            pltpu.make_async_copy(
                x_hbm.at[0, pl.ds(0, _R), :], xb.at[t % _NBUF],
                insem.at[t % _NBUF],
            ).wait()

        def fire_pin(s):
            pltpu.make_async_copy(
                pos_hbm.at[pl.ds(s * _R, _R), :], pb.at[s % 2], psem.at[s % 2]
            ).start()

        def wait_pin(s):
            pltpu.make_async_copy(
                pos_hbm.at[pl.ds(0, _R), :], pb.at[s % 2], psem.at[s % 2]
            ).wait()

        def fire_out(t):
            s, b = steps[t]
            pltpu.make_async_copy(
                xb.at[t % _NBUF], o_hbm.at[b, pl.ds(s * _R, _R), :],
                outsem.at[t % _NBUF],
            ).start()

        def wait_out(t):
            pltpu.make_async_copy(
                xb.at[t % _NBUF], o_hbm.at[0, pl.ds(0, _R), :],
                outsem.at[t % _NBUF],
            ).wait()

        fire_pin(0)
        fire_pin(1)
        for t in range(_LOOK):
            fire_xin(t)

        last_waited = -1
        for t in range(T):
            if t + _LOOK < T:
                if t - _NBUF + _LOOK >= 0:
                    wait_out(t - _NBUF + _LOOK)
                    last_waited = t - _NBUF + _LOOK
                fire_xin(t + _LOOK)
            s, b = steps[t]
            if b == 0:
                wait_pin(s)
            wait_xin(t)
            xb[t % _NBUF] = xb[t % _NBUF] + pb[s % 2]
            fire_out(t)
            if b == batch - 1 and s + 2 < s_ch:
                fire_pin(s + 2)

        for t in range(last_waited + 1, T):
            wait_out(t)

    return body


def kernel(x, pos_table):
    batch, seq, dim = x.shape
    body = _make_body(batch, seq, dim)
    return pl.pallas_call(
        body,
        in_specs=[
            pl.BlockSpec(memory_space=pl.ANY),
            pl.BlockSpec(memory_space=pl.ANY),
        ],
        out_specs=pl.BlockSpec(memory_space=pl.ANY),
        out_shape=jax.ShapeDtypeStruct((batch, seq, dim), x.dtype),
        scratch_shapes=[
            pltpu.VMEM((_NBUF, _R, dim), jnp.float32),
            pltpu.VMEM((2, _R, dim), jnp.float32),
            pltpu.SemaphoreType.DMA((_NBUF,)),
            pltpu.SemaphoreType.DMA((2,)),
            pltpu.SemaphoreType.DMA((_NBUF,)),
        ],
    )(x, pos_table)


# speedup vs baseline: 1.0075x; 1.0019x over previous
import jax
import jax.numpy as jnp
from jax.experimental import pallas as pl
from jax.experimental.pallas import tpu as pltpu

_R = 2048
_NBUF = 5
_LOOK = 2


def _make_body(batch, seq, dim):
    s_ch = seq // _R
    steps = [(s, b) for s in range(s_ch) for b in range(batch)]
    T = len(steps)

    def body(x_hbm, pos_hbm, o_hbm, xb, pb, insem, psem, outsem):
        def fire_xin(t):
            s, b = steps[t]
            pltpu.make_async_copy(
                x_hbm.at[b, pl.ds(s * _R, _R), :], xb.at[t % _NBUF],
                insem.at[t % _NBUF],
            ).start()

        def wait_xin(t):
            pltpu.make_async_copy(
                x_hbm.at[0, pl.ds(0, _R), :], xb.at[t % _NBUF],
                insem.at[t % _NBUF],
            ).wait()

        def fire_pin(s):
            pltpu.make_async_copy(
                pos_hbm.at[pl.ds(s * _R, _R), :], pb.at[s % 2], psem.at[s % 2]
            ).start()

        def wait_pin(s):
            pltpu.make_async_copy(
                pos_hbm.at[pl.ds(0, _R), :], pb.at[s % 2], psem.at[s % 2]
            ).wait()

        def fire_out(t):
            s, b = steps[t]
            pltpu.make_async_copy(
                xb.at[t % _NBUF], o_hbm.at[b, pl.ds(s * _R, _R), :],
                outsem.at[t % _NBUF],
            ).start()

        def wait_out(t):
            pltpu.make_async_copy(
                xb.at[t % _NBUF], o_hbm.at[0, pl.ds(0, _R), :],
                outsem.at[t % _NBUF],
            ).wait()

        fire_pin(0)
        fire_pin(1)
        for t in range(_LOOK):
            fire_xin(t)

        last_waited = -1
        for t in range(T):
            if t + _LOOK < T:
                if t - _NBUF + _LOOK >= 0:
                    wait_out(t - _NBUF + _LOOK)
                    last_waited = t - _NBUF + _LOOK
                fire_xin(t + _LOOK)
            s, b = steps[t]
            if b == 0:
                wait_pin(s)
            wait_xin(t)
            xb[t % _NBUF] = xb[t % _NBUF] + pb[s % 2]
            fire_out(t)
            if b == batch - 1 and s + 2 < s_ch:
                fire_pin(s + 2)

        for t in range(last_waited + 1, T):
            wait_out(t)

    return body


def kernel(x, pos_table):
    batch, seq, dim = x.shape
    body = _make_body(batch, seq, dim)
    return pl.pallas_call(
        body,
        in_specs=[
            pl.BlockSpec(memory_space=pl.ANY),
            pl.BlockSpec(memory_space=pl.ANY),
        ],
        out_specs=pl.BlockSpec(memory_space=pl.ANY),
        out_shape=jax.ShapeDtypeStruct((batch, seq, dim), x.dtype),
        scratch_shapes=[
            pltpu.VMEM((_NBUF, _R, dim), jnp.float32),
            pltpu.VMEM((2, _R, dim), jnp.float32),
            pltpu.SemaphoreType.DMA((_NBUF,)),
            pltpu.SemaphoreType.DMA((2,)),
            pltpu.SemaphoreType.DMA((_NBUF,)),
        ],
    )(x, pos_table)
